# Initial kernel scaffold; baseline (speedup 1.0000x reference)
import jax
import jax.numpy as jnp
from jax.experimental import pallas as pl


def kernel(dep_x, dep_edge_index, dep_batch, obj_boxes, obj_labels, obj_edge_index, obj_batch, Wt0, bt0, Wo0, bo0, Wk, Wq, Wvd, Wvo, Wt1, bt1, Wt2, bt2, Ws_rel, Wo_rel, Wbs, Wbo, F1, fb1, F2, fb2):
    raise NotImplementedError("write your pallas kernel here")


# speedup vs baseline: 4.5099x; 4.5099x over previous
import functools
import math

import jax
import jax.numpy as jnp
from jax import lax
from jax.experimental import pallas as pl
from jax.experimental.pallas import tpu as pltpu
from jax.experimental.pallas import tpu_sc as plsc

F32 = jnp.float32
I32 = jnp.int32

NC, NS = 2, 16
NW = NC * NS

ND, D = 10000, 128
NDP = 10240
ED = 320000
NCH_D = 80
NCH_D2 = 160
EDP = NW * NCH_D * 128
NO = 1024
NCH_O = 4
EO = 16384
NB = 64

DEP_TILE = 512
NT = NDP // DEP_TILE
RPT = NDP // NS
RPT_O = NO // NS

_MESH = dict(core_axis_name="c", subcore_axis_name="s",
             num_cores=NC, num_subcores=NS)


def _wid():
    return lax.axis_index("c") * NS + lax.axis_index("s")


def _take16(v, idx):
    return v.at[idx].get(mode="promise_in_bounds")


def _lane_sum16(v, iota16):
    for k in (8, 4, 2, 1):
        v = v + _take16(v, jnp.bitwise_xor(iota16, k))
    return v


def _sc_deg_dep(dst3, zeros16, ones16):
    @functools.partial(
        pl.kernel,
        out_type=jax.ShapeDtypeStruct((NC, NDP, 16), F32),
        mesh=plsc.VectorSubcoreMesh(**_MESH),
        scratch_types=[
            pltpu.VMEM((NCH_D, 128), I32),
            pltpu.VMEM((128, 16), F32),
            pltpu.VMEM_SHARED((NDP, 16), F32),
        ],
    )
    def k(dst_h, z_h, o_h, out_h, didx, ones_v, acc):
        cid = lax.axis_index("c")
        tid = lax.axis_index("s")
        for kk in range(RPT // 128):
            pltpu.sync_copy(z_h, acc.at[pl.ds(tid * RPT + kk * 128, 128)])
        pltpu.sync_copy(o_h, ones_v)
        pltpu.sync_copy(dst_h.at[_wid()], didx)
        plsc.subcore_barrier()

        def body(c, carry):
            pltpu.sync_copy(ones_v, acc.at[didx.at[c]], add=True)
            return carry

        lax.fori_loop(0, NCH_D, body, 0)
        plsc.subcore_barrier()
        for kk in range(RPT // 128):
            r = tid * RPT + kk * 128
            pltpu.sync_copy(acc.at[pl.ds(r, 128)], out_h.at[cid, pl.ds(r, 128)])

    return k(dst3, zeros16, ones16)


HALF = NDP // NC
ACC_D = 6144


def _sc_conv_dep(hp, src3, dst3, zeros128):
    @functools.partial(
        pl.kernel,
        out_type=jax.ShapeDtypeStruct((NDP, D), F32),
        mesh=plsc.VectorSubcoreMesh(**_MESH),
        scratch_types=[
            pltpu.VMEM((NCH_D2, 128), I32),
            pltpu.VMEM((NCH_D2, 128), I32),
            pltpu.VMEM((128, D), F32),
            pltpu.VMEM((128, D), F32),
            pltpu.VMEM_SHARED((ACC_D, D), F32),
            pltpu.SemaphoreType.DMA,
            pltpu.SemaphoreType.DMA,
        ],
    )
    def k(hp_h, src_h, dst_h, z_h, out_h, sidx, didx, buf0, buf1, acc,
          sem0, sem1):
        cid = lax.axis_index("c")
        tid = lax.axis_index("s")
        for kk in range(ACC_D // NS // 128):
            r = tid * (ACC_D // NS) + kk * 128
            pltpu.sync_copy(z_h, acc.at[pl.ds(r, 128)])
        pltpu.sync_copy(src_h.at[tid], sidx)
        pltpu.sync_copy(dst_h.at[tid], didx)
        base = cid * HALF

        def remap(c, carry):
            for q in range(8):
                sl = pl.ds(q * 16, 16)
                lv = didx[c, sl] - base
                ok = (lv >= 0) & (lv < HALF)
                didx[c, sl] = jnp.where(ok, lv, HALF)
            return carry

        lax.fori_loop(0, NCH_D2, remap, 0)
        plsc.subcore_barrier()

        pltpu.async_copy(hp_h.at[sidx.at[0]], buf0, sem0)

        def body(kk, carry):
            c0 = kk * 2
            pltpu.make_async_copy(hp_h.at[sidx.at[c0]], buf0, sem0).wait()
            pltpu.async_copy(hp_h.at[sidx.at[c0 + 1]], buf1, sem1)
            pltpu.sync_copy(buf0, acc.at[didx.at[c0]], add=True)
            pltpu.make_async_copy(hp_h.at[sidx.at[c0 + 1]], buf1, sem1).wait()

            @pl.when(kk < NCH_D2 // 2 - 1)
            def _():
                pltpu.async_copy(hp_h.at[sidx.at[c0 + 2]], buf0, sem0)

            pltpu.sync_copy(buf1, acc.at[didx.at[c0 + 1]], add=True)
            return carry

        lax.fori_loop(0, NCH_D2 // 2, body, 0)
        plsc.subcore_barrier()
        for (off, nrow) in ((0, 128), (128, 128), (256, 64)):
            r = tid * (HALF // NS) + off
            pltpu.sync_copy(acc.at[pl.ds(r, nrow)],
                            out_h.at[pl.ds(base + r, nrow)])

    return k(hp, src3, dst3, zeros128)


def _sc_obj_prep(labels, ws_rel, wo_rel, wo0, bw, bo):
    RP = NO // NW

    @functools.partial(
        pl.kernel,
        out_type=[jax.ShapeDtypeStruct((NO, 128), F32),
                  jax.ShapeDtypeStruct((NO, 128), F32),
                  jax.ShapeDtypeStruct((NO, D), F32)],
        mesh=plsc.VectorSubcoreMesh(**_MESH),
        scratch_types=[
            pltpu.VMEM((RP,), I32),
            pltpu.VMEM((RP, 128), F32),
            pltpu.VMEM((RP, 128), F32),
            pltpu.VMEM((RP, D), F32),
            pltpu.SemaphoreType.DMA,
        ],
    )
    def k(lab_h, ws_h, wo_h, wo0_h, bw_h, bo_h, subj_h, objf_h, hobj_h,
          lab, a, b, hv, sem):
        base = _wid() * RP
        pltpu.sync_copy(lab_h.at[pl.ds(base, RP)], lab)

        for (tab, lin, dst) in ((ws_h, bw_h, subj_h), (wo_h, bo_h, objf_h)):
            pltpu.async_copy(tab.at[lab], a, sem).wait()
            pltpu.sync_copy(lin.at[pl.ds(base, RP)], b)

            def rbody(r, carry):
                for q in range(4):
                    sl = pl.ds(q * 16, 16)
                    a[r, sl] = a[r, sl] + b[r, sl]
                return carry

            lax.fori_loop(0, RP, rbody, 0)
            pltpu.sync_copy(a, dst.at[pl.ds(base, RP)])

        pltpu.async_copy(wo0_h.at[lab], hv, sem).wait()
        pltpu.sync_copy(hv, hobj_h.at[pl.ds(base, RP)])

    return k(labels, ws_rel, wo_rel, wo0, bw, bo)


def _sc_relpn(subj, objf, src3, dst3, zeros16):
    @functools.partial(
        pl.kernel,
        out_type=[jax.ShapeDtypeStruct((NW, NCH_O, 128), F32),
                  jax.ShapeDtypeStruct((NC, NO, 16), F32)],
        mesh=plsc.VectorSubcoreMesh(**_MESH),
        scratch_types=[
            pltpu.VMEM((NCH_O, 128), I32),
            pltpu.VMEM((NCH_O, 128), I32),
            pltpu.VMEM((128, 128), F32),
            pltpu.VMEM((128, 128), F32),
            pltpu.VMEM((128,), F32),
            pltpu.VMEM((128, 16), F32),
            pltpu.VMEM_SHARED((NO, 16), F32),
            pltpu.SemaphoreType.DMA,
            pltpu.SemaphoreType.DMA,
        ],
    )
    def k(subj_h, objf_h, src_h, dst_h, z_h, ew_h, deg_h,
          sidx, didx, sbuf, obuf, ewbuf, colbuf, acc, sem0, sem1):
        cid = lax.axis_index("c")
        tid = lax.axis_index("s")
        wid = _wid()
        pltpu.sync_copy(z_h.at[pl.ds(0, RPT_O)], acc.at[pl.ds(tid * RPT_O, RPT_O)])
        pltpu.sync_copy(z_h, colbuf)
        pltpu.sync_copy(src_h.at[wid], sidx)
        pltpu.sync_copy(dst_h.at[wid], didx)
        plsc.subcore_barrier()

        iota16 = lax.iota(I32, 16)
        for c in range(NCH_O):
            d1 = pltpu.async_copy(subj_h.at[sidx.at[c]], sbuf, sem0)
            d2 = pltpu.async_copy(objf_h.at[didx.at[c]], obuf, sem1)
            d1.wait()
            d2.wait()

            def gbody(g, carry):
                def jbody(j, accv):
                    e = g * 16 + j
                    dv = sbuf[e, pl.ds(0, 16)] * obuf[e, pl.ds(0, 16)]
                    for q in range(1, 4):
                        sl = pl.ds(q * 16, 16)
                        dv = dv + sbuf[e, sl] * obuf[e, sl]
                    tot = _lane_sum16(dv, iota16)
                    sig = 1.0 / (1.0 + jnp.exp(-tot))
                    colbuf[e, pl.ds(0, 16)] = jnp.where(iota16 == 0, sig, 0.0)
                    return jnp.where(iota16 == j, sig, accv)

                ewv = lax.fori_loop(0, 16, jbody, jnp.zeros((16,), F32))
                ewbuf[pl.ds(g * 16, 16)] = ewv
                return carry

            lax.fori_loop(0, 8, gbody, 0)
            pltpu.sync_copy(ewbuf, ew_h.at[wid, c])
            pltpu.sync_copy(colbuf, acc.at[didx.at[c]], add=True)

        plsc.subcore_barrier()
        r = tid * RPT_O
        pltpu.sync_copy(acc.at[pl.ds(r, RPT_O)], deg_h.at[cid, pl.ds(r, RPT_O)])

    return k(subj, objf, src3, dst3, zeros16)


def _sc_conv_obj(hp, src3, dst3, ew3, zeros128):
    @functools.partial(
        pl.kernel,
        out_type=jax.ShapeDtypeStruct((NC, NO, D), F32),
        mesh=plsc.VectorSubcoreMesh(**_MESH),
        scratch_types=[
            pltpu.VMEM((NCH_O, 128), I32),
            pltpu.VMEM((NCH_O, 128), I32),
            pltpu.VMEM((NCH_O, 128), F32),
            pltpu.VMEM((128, D), F32),
            pltpu.VMEM_SHARED((NO, D), F32),
            pltpu.SemaphoreType.DMA,
        ],
    )
    def k(hp_h, src_h, dst_h, ew_h, z_h, out_h, sidx, didx, ewv, buf, acc, sem):
        cid = lax.axis_index("c")
        tid = lax.axis_index("s")
        wid = _wid()
        pltpu.sync_copy(z_h.at[pl.ds(0, RPT_O)], acc.at[pl.ds(tid * RPT_O, RPT_O)])
        pltpu.sync_copy(src_h.at[wid], sidx)
        pltpu.sync_copy(dst_h.at[wid], didx)
        pltpu.sync_copy(ew_h.at[wid], ewv)
        plsc.subcore_barrier()

        iota16 = lax.iota(I32, 16)
        for c in range(NCH_O):
            pltpu.async_copy(hp_h.at[sidx.at[c]], buf, sem).wait()

            def gbody(g, carry):
                ew16 = ewv[c, pl.ds(g * 16, 16)]

                def jbody(j, carry2):
                    w = _take16(ew16, jnp.full((16,), 0, I32) + j)
                    e = g * 16 + j
                    for q in range(8):
                        sl = pl.ds(q * 16, 16)
                        buf[e, sl] = buf[e, sl] * w
                    return carry2

                lax.fori_loop(0, 16, jbody, 0)
                return carry

            lax.fori_loop(0, 8, gbody, 0)
            pltpu.sync_copy(buf, acc.at[didx.at[c]], add=True)

        plsc.subcore_barrier()
        r = tid * RPT_O
        pltpu.sync_copy(acc.at[pl.ds(r, RPT_O)], out_h.at[cid, pl.ds(r, RPT_O)])

    return k(hp, src3, dst3, ew3, zeros128)


def _dot(a, b):
    return jnp.dot(a, b, preferred_element_type=F32)


def _tc_prep(dep_x_pad, wt0, boxes, wbs, wbo):
    def body(x_ref, w_ref, bx_ref, wbs_ref, wbo_ref, h_ref, bw_ref, bo_ref):
        i = pl.program_id(0)
        h_ref[...] = _dot(x_ref[...], w_ref[...])

        @pl.when(i == 0)
        def _():
            bw_ref[...] = _dot(bx_ref[...], wbs_ref[...])
            bo_ref[...] = _dot(bx_ref[...], wbo_ref[...])

    return pl.pallas_call(
        body,
        grid=(NT,),
        in_specs=[pl.BlockSpec((DEP_TILE, D), lambda i: (i, 0)),
                  pl.BlockSpec((D, D), lambda i: (0, 0)),
                  pl.BlockSpec((NO, 4), lambda i: (0, 0)),
                  pl.BlockSpec((4, 128), lambda i: (0, 0)),
                  pl.BlockSpec((4, 128), lambda i: (0, 0))],
        out_specs=[pl.BlockSpec((DEP_TILE, D), lambda i: (i, 0)),
                   pl.BlockSpec((NO, 128), lambda i: (0, 0)),
                   pl.BlockSpec((NO, 128), lambda i: (0, 0))],
        out_shape=[jax.ShapeDtypeStruct((NDP, D), F32),
                   jax.ShapeDtypeStruct((NO, 128), F32),
                   jax.ShapeDtypeStruct((NO, 128), F32)],
    )(dep_x_pad, wt0, boxes, wbs, wbo)


def _tc_glue_dep0(degp, h0):
    def body(dg_ref, h_ref, dinv_ref, hp_ref):
        i = pl.program_id(0)
        deg = dg_ref[0, :, 0:1] + dg_ref[1, :, 0:1] + 1.0
        dinv = lax.rsqrt(deg)
        rows = i * DEP_TILE + lax.broadcasted_iota(I32, (DEP_TILE, 1), 0)
        dinv_ref[...] = dinv
        hp_ref[...] = jnp.where(rows < ND, dinv * h_ref[...], 0.0)

    return pl.pallas_call(
        body,
        grid=(NT,),
        in_specs=[pl.BlockSpec((NC, DEP_TILE, 16), lambda i: (0, i, 0)),
                  pl.BlockSpec((DEP_TILE, D), lambda i: (i, 0))],
        out_specs=[pl.BlockSpec((DEP_TILE, 1), lambda i: (i, 0)),
                   pl.BlockSpec((DEP_TILE, D), lambda i: (i, 0))],
        out_shape=[jax.ShapeDtypeStruct((NDP, 1), F32),
                   jax.ShapeDtypeStruct((NDP, D), F32)],
    )(degp, h0)


def _tc_glue_obj0(degp, hobj):
    def body(dg_ref, h_ref, dinv_ref, hp_ref):
        deg = dg_ref[0, :, 0:1] + dg_ref[1, :, 0:1] + 1.0
        dinv = lax.rsqrt(deg)
        dinv_ref[...] = dinv
        hp_ref[...] = dinv * h_ref[...]

    return pl.pallas_call(
        body,
        in_specs=[pl.BlockSpec((NC, NO, 16), lambda: (0, 0, 0)),
                  pl.BlockSpec((NO, D), lambda: (0, 0))],
        out_specs=[pl.BlockSpec((NO, 1), lambda: (0, 0)),
                   pl.BlockSpec((NO, D), lambda: (0, 0))],
        out_shape=[jax.ShapeDtypeStruct((NO, 1), F32),
                   jax.ShapeDtypeStruct((NO, D), F32)],
    )(degp, hobj)


def _tc_biatt(pd, h0, dinv_d, bt0, po, hob, dinv_o, bo0, wk, wq, wvd, wvo):
    scale = 1.0 / math.sqrt(float(D))

    def body(pd_ref, h0_ref, dinv_ref, bt0_ref, po_ref, hob_ref, dinvo_ref,
             bo0_ref, wk_ref, wq_ref, wvd_ref, wvo_ref,
             depo_ref, objo_ref,
             q_s, vo_s, m_s, l_s, acc_s):
        i = pl.program_id(0)

        @pl.when(i == 0)
        def _():
            dvo = dinvo_ref[...]
            oh = (dvo * (po_ref[0] + po_ref[1]) + (dvo * dvo) * hob_ref[...]
                  + bo0_ref[...])
            for h in range(3):
                q_s[h] = _dot(oh, wq_ref[h])
                vo_s[h] = _dot(oh, wvo_ref[h])
            m_s[...] = jnp.full((3, NO), -1e30, F32)
            l_s[...] = jnp.zeros((3, NO), F32)
            acc_s[...] = jnp.zeros((3, D, NO), F32)

        dv = dinv_ref[...]
        dep_t = dv * pd_ref[...] + (dv * dv) * h0_ref[...] + bt0_ref[...]
        rows = i * DEP_TILE + lax.broadcasted_iota(I32, (DEP_TILE, 1), 0)
        valid = rows < ND

        dep_acc = jnp.zeros((DEP_TILE, D), F32)
        for h in range(3):
            kh = _dot(dep_t, wk_ref[h])
            vdh = _dot(dep_t, wvd_ref[h])
            a = lax.dot_general(kh, q_s[h], (((1,), (1,)), ((), ()))) * scale
            a = jnp.where(valid, a, -1e30)
            rmax = jnp.max(a, axis=1, keepdims=True)
            p = jnp.exp(a - rmax)
            rsum = jnp.sum(p, axis=1, keepdims=True)
            dep_acc = dep_acc + _dot(p, vo_s[h]) / rsum
            m_old = m_s[h:h + 1, :]
            m_new = jnp.maximum(m_old, jnp.max(a, axis=0, keepdims=True))
            alpha = jnp.exp(m_old - m_new)
            e = jnp.exp(a - m_new)
            l_s[h:h + 1, :] = l_s[h:h + 1, :] * alpha + jnp.sum(e, axis=0, keepdims=True)
            acc_s[h] = acc_s[h] * alpha + lax.dot_general(
                vdh, e, (((0,), (0,)), ((), ())))
            m_s[h:h + 1, :] = m_new

        depo_ref[...] = dep_acc * (1.0 / 3.0)

        @pl.when(i == NT - 1)
        def _():
            s = (acc_s[0] / l_s[0:1, :] + acc_s[1] / l_s[1:2, :]
                 + acc_s[2] / l_s[2:3, :])
            objo_ref[...] = jnp.transpose(s) * (1.0 / 3.0)

    return pl.pallas_call(
        body,
        grid=(NT,),
        in_specs=[pl.BlockSpec((DEP_TILE, D), lambda i: (i, 0)),
                  pl.BlockSpec((DEP_TILE, D), lambda i: (i, 0)),
                  pl.BlockSpec((DEP_TILE, 1), lambda i: (i, 0)),
                  pl.BlockSpec((1, D), lambda i: (0, 0)),
                  pl.BlockSpec((NC, NO, D), lambda i: (0, 0, 0)),
                  pl.BlockSpec((NO, D), lambda i: (0, 0)),
                  pl.BlockSpec((NO, 1), lambda i: (0, 0)),
                  pl.BlockSpec((1, D), lambda i: (0, 0)),
                  pl.BlockSpec((3, D, D), lambda i: (0, 0, 0)),
                  pl.BlockSpec((3, D, D), lambda i: (0, 0, 0)),
                  pl.BlockSpec((3, D, D), lambda i: (0, 0, 0)),
                  pl.BlockSpec((3, D, D), lambda i: (0, 0, 0))],
        out_specs=[pl.BlockSpec((DEP_TILE, D), lambda i: (i, 0)),
                   pl.BlockSpec((NO, D), lambda i: (0, 0))],
        out_shape=[jax.ShapeDtypeStruct((NDP, D), F32),
                   jax.ShapeDtypeStruct((NO, D), F32)],
        scratch_shapes=[pltpu.VMEM((3, NO, D), F32),
                        pltpu.VMEM((3, NO, D), F32),
                        pltpu.VMEM((3, NO), F32),
                        pltpu.VMEM((3, NO), F32),
                        pltpu.VMEM((3, D, NO), F32)],
    )(pd, h0, dinv_d, bt0, po, hob, dinv_o, bo0, wk, wq, wvd, wvo)


def _tc_glue1(dep_out, wt1, dinv_d):
    def body(x_ref, w_ref, dinv_ref, h_ref, hp_ref):
        i = pl.program_id(0)
        h1 = _dot(x_ref[...], w_ref[...])
        rows = i * DEP_TILE + lax.broadcasted_iota(I32, (DEP_TILE, 1), 0)
        h_ref[...] = h1
        hp_ref[...] = jnp.where(rows < ND, dinv_ref[...] * h1, 0.0)

    return pl.pallas_call(
        body,
        grid=(NT,),
        in_specs=[pl.BlockSpec((DEP_TILE, D), lambda i: (i, 0)),
                  pl.BlockSpec((D, D), lambda i: (0, 0)),
                  pl.BlockSpec((DEP_TILE, 1), lambda i: (i, 0))],
        out_specs=[pl.BlockSpec((DEP_TILE, D), lambda i: (i, 0)),
                   pl.BlockSpec((DEP_TILE, D), lambda i: (i, 0))],
        out_shape=[jax.ShapeDtypeStruct((NDP, D), F32),
                   jax.ShapeDtypeStruct((NDP, D), F32)],
    )(dep_out, wt1, dinv_d)


def _tc_glue2(p, h_prev, dinv_d, b_prev, w_next):
    def body(p_ref, h_ref, dinv_ref, b_ref, w_ref, hn_ref, hp_ref):
        i = pl.program_id(0)
        dv = dinv_ref[...]
        out = dv * p_ref[...] + (dv * dv) * h_ref[...] + b_ref[...]
        hn = _dot(out, w_ref[...])
        rows = i * DEP_TILE + lax.broadcasted_iota(I32, (DEP_TILE, 1), 0)
        hn_ref[...] = hn
        hp_ref[...] = jnp.where(rows < ND, dv * hn, 0.0)

    return pl.pallas_call(
        body,
        grid=(NT,),
        in_specs=[pl.BlockSpec((DEP_TILE, D), lambda i: (i, 0)),
                  pl.BlockSpec((DEP_TILE, D), lambda i: (i, 0)),
                  pl.BlockSpec((DEP_TILE, 1), lambda i: (i, 0)),
                  pl.BlockSpec((1, D), lambda i: (0, 0)),
                  pl.BlockSpec((D, D), lambda i: (0, 0))],
        out_specs=[pl.BlockSpec((DEP_TILE, D), lambda i: (i, 0)),
                   pl.BlockSpec((DEP_TILE, D), lambda i: (i, 0))],
        out_shape=[jax.ShapeDtypeStruct((NDP, D), F32),
                   jax.ShapeDtypeStruct((NDP, D), F32)],
    )(p, h_prev, dinv_d, b_prev, w_next)


def _tc_final(p2, h2, dinv_d, bt2, batch_d, obj_h, batch_o, f1, fb1, f2, fb2):
    hid = f1.shape[1]
    acls = f2.shape[1]

    def body(p_ref, h_ref, dinv_ref, b_ref, bd_ref, oh_ref, bo_ref,
             f1_ref, fb1_ref, f2_ref, fb2_ref, out_ref, sums_s, cnt_s):
        i = pl.program_id(0)

        @pl.when(i == 0)
        def _():
            sums_s[...] = jnp.zeros((NB, D), F32)
            cnt_s[...] = jnp.zeros((1, NB), F32)

        @pl.when(i < NT)
        def _():
            dv = dinv_ref[...]
            dep_h = dv * p_ref[...] + (dv * dv) * h_ref[...] + b_ref[...]
            cols = lax.broadcasted_iota(I32, (1, NB), 1)
            mask = (bd_ref[...] == cols).astype(F32)
            sums_s[...] = sums_s[...] + lax.dot_general(
                mask, dep_h, (((0,), (0,)), ((), ())))
            cnt_s[...] = cnt_s[...] + jnp.sum(mask, axis=0, keepdims=True)

        @pl.when(i == NT)
        def _():
            cols = lax.broadcasted_iota(I32, (1, NB), 1)
            masko = (bo_ref[...] == cols).astype(F32)
            osum = lax.dot_general(masko, oh_ref[...], (((0,), (0,)), ((), ())))
            ocnt = jnp.sum(masko, axis=0, keepdims=True)
            dep_p = sums_s[...] / jnp.clip(cnt_s[...], 1.0).reshape(NB, 1)
            obj_p = osum / jnp.clip(ocnt, 1.0).reshape(NB, 1)
            fused = jnp.concatenate([dep_p, obj_p], axis=1)
            hh = _dot(fused, f1_ref[...]) + fb1_ref[...]
            lg = _dot(hh, f2_ref[...]) + fb2_ref[...]
            mx = jnp.max(lg, axis=1, keepdims=True)
            lse = mx + jnp.log(jnp.sum(jnp.exp(lg - mx), axis=1, keepdims=True))
            out_ref[...] = lg - lse

    def dep_idx(i):
        return jnp.minimum(i, NT - 1)

    return pl.pallas_call(
        body,
        grid=(NT + 1,),
        in_specs=[pl.BlockSpec((DEP_TILE, D), lambda i: (dep_idx(i), 0)),
                  pl.BlockSpec((DEP_TILE, D), lambda i: (dep_idx(i), 0)),
                  pl.BlockSpec((DEP_TILE, 1), lambda i: (dep_idx(i), 0)),
                  pl.BlockSpec((1, D), lambda i: (0, 0)),
                  pl.BlockSpec((DEP_TILE, 1), lambda i: (dep_idx(i), 0)),
                  pl.BlockSpec((NO, D), lambda i: (0, 0)),
                  pl.BlockSpec((NO, 1), lambda i: (0, 0)),
                  pl.BlockSpec((2 * D, hid), lambda i: (0, 0)),
                  pl.BlockSpec((1, hid), lambda i: (0, 0)),
                  pl.BlockSpec((hid, acls), lambda i: (0, 0)),
                  pl.BlockSpec((1, acls), lambda i: (0, 0))],
        out_specs=pl.BlockSpec((NB, acls), lambda i: (0, 0)),
        out_shape=jax.ShapeDtypeStruct((NB, acls), F32),
        scratch_shapes=[pltpu.VMEM((NB, D), F32),
                        pltpu.VMEM((1, NB), F32)],
    )(p2, h2, dinv_d, bt2, batch_d, obj_h, batch_o, f1, fb1, f2, fb2)


def kernel(dep_x, dep_edge_index, dep_batch, obj_boxes, obj_labels,
           obj_edge_index, obj_batch, Wt0, bt0, Wo0, bo0, Wk, Wq, Wvd, Wvo,
           Wt1, bt1, Wt2, bt2, Ws_rel, Wo_rel, Wbs, Wbo, F1, fb1, F2, fb2):
    pad_e = EDP - ED
    pad_idx = jnp.full((pad_e,), NDP - 1, I32)
    src_flat = jnp.concatenate([dep_edge_index[0].astype(I32), pad_idx])
    dst_flat = jnp.concatenate([dep_edge_index[1].astype(I32), pad_idx])
    dst3 = dst_flat.reshape(NW, NCH_D, 128)
    src3c = src_flat.reshape(NS, NCH_D2, 128)
    dst3c = dst_flat.reshape(NS, NCH_D2, 128)
    osrc3 = obj_edge_index[0].astype(I32).reshape(NW, NCH_O, 128)
    odst3 = obj_edge_index[1].astype(I32).reshape(NW, NCH_O, 128)
    dep_x_pad = jnp.pad(dep_x, ((0, NDP - ND), (0, 0)))
    batch_d = jnp.pad(dep_batch.astype(I32), (0, NDP - ND),
                      constant_values=NB).reshape(NDP, 1)
    batch_o = obj_batch.astype(I32).reshape(NO, 1)
    zeros16 = jnp.zeros((128, 16), F32)
    ones16 = jnp.ones((128, 16), F32)
    zeros128 = jnp.zeros((128, 128), F32)
    bt0r, bo0r = bt0.reshape(1, D), bo0.reshape(1, D)
    bt1r, bt2r = bt1.reshape(1, D), bt2.reshape(1, D)
    fb1r, fb2r = fb1.reshape(1, -1), fb2.reshape(1, -1)

    wbs_p = jnp.pad(Wbs, ((0, 0), (0, 64)))
    wbo_p = jnp.pad(Wbo, ((0, 0), (0, 64)))
    ws_rel_p = jnp.pad(Ws_rel, ((0, 0), (0, 64)))
    wo_rel_p = jnp.pad(Wo_rel, ((0, 0), (0, 64)))
    h0, bw, bo_lin = _tc_prep(dep_x_pad, Wt0, obj_boxes, wbs_p, wbo_p)
    degp_d = _sc_deg_dep(dst3, zeros16, ones16)
    subj, objf, hobj = _sc_obj_prep(obj_labels.astype(I32), ws_rel_p, wo_rel_p,
                                    Wo0, bw, bo_lin)
    ew3, degp_o = _sc_relpn(subj, objf, osrc3, odst3, zeros16)

    dinv_d, hp0 = _tc_glue_dep0(degp_d, h0)
    dinv_o, hpo = _tc_glue_obj0(degp_o, hobj)

    pd0 = _sc_conv_dep(hp0, src3c, dst3c, zeros128)
    po0 = _sc_conv_obj(hpo, osrc3, odst3, ew3, zeros128)
    dep_out, obj_h = _tc_biatt(pd0, h0, dinv_d, bt0r, po0, hobj, dinv_o,
                               bo0r, Wk, Wq, Wvd, Wvo)

    h1, hp1 = _tc_glue1(dep_out, Wt1, dinv_d)
    pd1 = _sc_conv_dep(hp1, src3c, dst3c, zeros128)
    h2, hp2 = _tc_glue2(pd1, h1, dinv_d, bt1r, Wt2)
    pd2 = _sc_conv_dep(hp2, src3c, dst3c, zeros128)

    return _tc_final(pd2, h2, dinv_d, bt2r, batch_d, obj_h, batch_o,
                     F1, fb1r, F2, fb2r)
